# baseline (speedup 1.0000x reference)
import jax
import jax.numpy as jnp
from jax import lax
from jax.experimental import pallas as pl
from jax.experimental.pallas import tpu as pltpu
from jax.experimental.pallas import tpu_sc as plsc

N = 10000
C = 128
H = 64
E = 320000
NSUB = 16
EPT = E // NSUB
CH = 80
NCHUNK = EPT // CH
BN = 1000
NB = N // BN


def _dot(a, b):
    return jnp.dot(a, b, preferred_element_type=jnp.float32)


def _dense_half(X, posp_ref, wpos_ref, bpos_ref, wsrc_ref, wlin_ref):
    P = _dot(posp_ref[...], wpos_ref[0])
    Pb = P + bpos_ref[0]
    S = _dot(X, wsrc_ref[0]) + P
    V = _dot(X, wlin_ref[0]) - P
    return Pb, S, V


def _phase1_out(i, posp_ref, wpos_ref, bpos_ref, sv_s, mn_s, t_ref, pb_ref):
    P = _dot(posp_ref[...], wpos_ref[0])
    pb_ref[0] = P + bpos_ref[0]
    SV = sv_s[pl.ds(i * BN, BN), :]
    Es = jnp.exp(mn_s[0:1, :H] - SV[:, :H])
    t_ref[0] = jnp.concatenate([Es, Es * SV[:, H:]], axis=1)


def _prep_body(x_ref, posp_ref, wpos_ref, bpos_ref, wsrc_ref, wlin_ref,
               t_ref, pb_ref, sv_s, mn_s):
    p = pl.program_id(1)
    i = pl.program_id(2)

    @pl.when(p == 0)
    def _():
        _, S, V = _dense_half(x_ref[...], posp_ref, wpos_ref, bpos_ref,
                              wsrc_ref, wlin_ref)
        sv_s[pl.ds(i * BN, BN), :] = jnp.concatenate([S, V], axis=1)
        cmn = jnp.min(S, axis=0, keepdims=True)

        @pl.when(i == 0)
        def _():
            mn_s[0:1, :H] = cmn

        @pl.when(i > 0)
        def _():
            mn_s[0:1, :H] = jnp.minimum(mn_s[0:1, :H], cmn)

    @pl.when(p == 1)
    def _():
        _phase1_out(i, posp_ref, wpos_ref, bpos_ref, sv_s, mn_s, t_ref,
                    pb_ref)


_PREP_SPECS = dict(
    grid=(2, 2, NB),
    out_specs=[
        pl.BlockSpec((1, BN, C), lambda h, p, i: (h, i, 0)),
        pl.BlockSpec((1, BN, H), lambda h, p, i: (h, i, 0)),
    ],
    out_shape=[
        jax.ShapeDtypeStruct((2, N, C), jnp.float32),
        jax.ShapeDtypeStruct((2, N, H), jnp.float32),
    ],
    scratch_shapes=[
        pltpu.VMEM((N, C), jnp.float32),
        pltpu.VMEM((8, C), jnp.float32),
    ],
    compiler_params=pltpu.CompilerParams(
        dimension_semantics=("arbitrary", "arbitrary", "arbitrary")),
)

_W_SPECS = [
    pl.BlockSpec((BN, 8), lambda h, p, i: (i, 0)),
    pl.BlockSpec((1, 8, H), lambda h, p, i: (h, 0, 0)),
    pl.BlockSpec((1, 1, H), lambda h, p, i: (h, 0, 0)),
    pl.BlockSpec((1, C, H), lambda h, p, i: (h, 0, 0)),
    pl.BlockSpec((1, C, H), lambda h, p, i: (h, 0, 0)),
]


def _prep(x, posp, wposp, bpos2, wsrc, wlin):
    return pl.pallas_call(
        _prep_body,
        in_specs=[pl.BlockSpec((BN, C), lambda h, p, i: (i, 0))] + _W_SPECS,
        **_PREP_SPECS,
    )(x, posp, wposp, bpos2, wsrc, wlin)


def _finalize_h(acc_ref, pb_ref):
    den = jnp.concatenate([acc_ref[0, :, :H], acc_ref[1, :, :H]], axis=1)
    num = jnp.concatenate([acc_ref[0, :, H:], acc_ref[1, :, H:]], axis=1)
    pbf = jnp.concatenate([pb_ref[0], pb_ref[1]], axis=1)
    hidden = jnp.where(den > 0.0, num / den + pbf, 0.0)
    return jnp.maximum(hidden, 0.0)


def _mid_body(acc_ref, pb1_ref, posp_ref, wpos_ref, bpos_ref, wsrc_ref,
              wlin_ref, t_ref, pb_ref, sv_s, mn_s):
    p = pl.program_id(1)
    i = pl.program_id(2)

    @pl.when(p == 0)
    def _():
        X = _finalize_h(acc_ref, pb1_ref)
        _, S, V = _dense_half(X, posp_ref, wpos_ref, bpos_ref, wsrc_ref,
                              wlin_ref)
        sv_s[pl.ds(i * BN, BN), :] = jnp.concatenate([S, V], axis=1)
        cmn = jnp.min(S, axis=0, keepdims=True)

        @pl.when(i == 0)
        def _():
            mn_s[0:1, :H] = cmn

        @pl.when(i > 0)
        def _():
            mn_s[0:1, :H] = jnp.minimum(mn_s[0:1, :H], cmn)

    @pl.when(p == 1)
    def _():
        _phase1_out(i, posp_ref, wpos_ref, bpos_ref, sv_s, mn_s, t_ref,
                    pb_ref)


def _mid(acc1, pb1, posp, wposp, bpos2, wsrc, wlin):
    return pl.pallas_call(
        _mid_body,
        in_specs=[
            pl.BlockSpec((2, BN, C), lambda h, p, i: (0, i, 0)),
            pl.BlockSpec((2, BN, H), lambda h, p, i: (0, i, 0)),
        ] + _W_SPECS,
        **_PREP_SPECS,
    )(acc1, pb1, posp, wposp, bpos2, wsrc, wlin)


def _head_body(acc_ref, pb2_ref, w1_ref, b1_ref, w2t_ref, b2_ref, out_ref):
    hidden = _finalize_h(acc_ref, pb2_ref)
    f = _dot(hidden, w1_ref[...])
    f = jnp.maximum(f + b1_ref[...], 0.0)
    out_ref[...] = (jnp.sum(f * w2t_ref[...], axis=1, keepdims=True)
                    + b2_ref[...])


def _head(acc2, pb2, fc1w, fc1b2, fc2wt, fc2b2):
    return pl.pallas_call(
        _head_body,
        grid=(NB,),
        in_specs=[
            pl.BlockSpec((2, BN, C), lambda i: (0, i, 0)),
            pl.BlockSpec((2, BN, H), lambda i: (0, i, 0)),
            pl.BlockSpec((C, H), lambda i: (0, 0)),
            pl.BlockSpec((1, H), lambda i: (0, 0)),
            pl.BlockSpec((1, H), lambda i: (0, 0)),
            pl.BlockSpec((1, 1), lambda i: (0, 0)),
        ],
        out_specs=pl.BlockSpec((BN, 1), lambda i: (i, 0)),
        out_shape=jax.ShapeDtypeStruct((N, 1), jnp.float32),
    )(acc2, pb2, fc1w, fc1b2, fc2wt, fc2b2)


def _edge_body(t_hbm, dst_hbm, src_hbm, z_hbm, acc_hbm, shared, dstv, idxv,
               rows0, rows1, sem0, sem1):
    c = lax.axis_index("c")
    s = lax.axis_index("s")
    coff = c * N
    rbase = s * 640

    @pl.when(s < 15)
    def _():
        pltpu.sync_copy(z_hbm.at[pl.ds(rbase, 640)],
                        shared.at[pl.ds(rbase, 640)])

    @pl.when(s == 15)
    def _():
        pltpu.sync_copy(z_hbm.at[pl.ds(9600, 400)],
                        shared.at[pl.ds(9600, 400)])

    plsc.subcore_barrier()

    def gather(j, rows, sem):
        return pltpu.async_copy(
            t_hbm.at[idxv.at[pl.ds(j * CH, CH)]], rows, sem)

    def wait(j, rows, sem):
        pltpu.make_async_copy(
            t_hbm.at[idxv.at[pl.ds(j * CH, CH)]], rows, sem).wait()

    def scatter(j, rows):
        pltpu.sync_copy(rows, shared.at[dstv.at[j]], add=True)

    NC2 = NCHUNK // 2
    NPAIR = NC2 // 2

    def stage_body(hh, carry):
        pltpu.sync_copy(dst_hbm.at[s, hh], dstv)
        pltpu.sync_copy(src_hbm.at[c, s, hh], idxv)

        gather(0, rows0, sem0)

        def pair_body(k, cc):
            j0 = 2 * k
            j1 = j0 + 1
            gather(j1, rows1, sem1)
            wait(j0, rows0, sem0)
            scatter(j0, rows0)

            @pl.when(k < NPAIR - 1)
            def _():
                gather(j0 + 2, rows0, sem0)

            wait(j1, rows1, sem1)
            scatter(j1, rows1)
            return cc

        lax.fori_loop(0, NPAIR, pair_body, 0)
        gather(NC2 - 1, rows0, sem0)
        wait(NC2 - 1, rows0, sem0)
        scatter(NC2 - 1, rows0)
        return carry

    lax.fori_loop(0, 2, stage_body, 0)
    plsc.subcore_barrier()

    @pl.when(s < 15)
    def _():
        pltpu.sync_copy(shared.at[pl.ds(rbase, 640)],
                        acc_hbm.at[pl.ds(coff + rbase, 640)])

    @pl.when(s == 15)
    def _():
        pltpu.sync_copy(shared.at[pl.ds(9600, 400)],
                        acc_hbm.at[pl.ds(coff + 9600, 400)])


def _edge(tstk, dst3, src2, zrows):
    f = pl.kernel(
        _edge_body,
        out_type=jax.ShapeDtypeStruct((2 * N, C), jnp.float32),
        mesh=plsc.VectorSubcoreMesh(core_axis_name="c", subcore_axis_name="s"),
        scratch_types=[
            pltpu.VMEM_SHARED((N, C), jnp.float32),
            pltpu.VMEM((NCHUNK // 2, CH), jnp.int32),
            pltpu.VMEM((EPT // 2,), jnp.int32),
            pltpu.VMEM((CH, C), jnp.float32),
            pltpu.VMEM((CH, C), jnp.float32),
            pltpu.SemaphoreType.DMA,
            pltpu.SemaphoreType.DMA,
        ],
    )
    return f(tstk, dst3, src2, zrows)


def kernel(x, pos, edge_index, batch, c1_Wpos, c1_bpos, c1_Wsrc, c1_Wdst,
           c1_Wlin, c2_Wpos, c2_bpos, c2_Wsrc, c2_Wdst, c2_Wlin, fc1_W,
           fc1_b, fc2_W, fc2_b):
    def _half(w):
        return jnp.stack([w[:, :H], w[:, H:]])

    src_r = edge_index[0].reshape(NSUB, 2, EPT // 2)
    src_a = jnp.stack([src_r, src_r + N])
    dst_a = edge_index[1].reshape(NSUB, 2, NCHUNK // 2, CH)
    posp = jnp.pad(pos, ((0, 0), (0, 5)))
    w1p = _half(jnp.pad(c1_Wpos, ((0, 5), (0, 0))))
    w2p = _half(jnp.pad(c2_Wpos, ((0, 5), (0, 0))))
    b1_2 = c1_bpos.reshape(2, 1, H)
    b2_2 = c2_bpos.reshape(2, 1, H)
    zrows = jnp.zeros((N, C), jnp.float32)

    t1, pb1 = _prep(x, posp, w1p, b1_2, _half(c1_Wsrc), _half(c1_Wlin))
    acc1 = _edge(t1.reshape(2 * N, C), dst_a, src_a, zrows)
    t2, pb2 = _mid(acc1.reshape(2, N, C), pb1, posp, w2p, b2_2,
                   _half(c2_Wsrc), _half(c2_Wlin))
    acc2 = _edge(t2.reshape(2 * N, C), dst_a, src_a, zrows)
    out = _head(acc2.reshape(2, N, C), pb2, fc1_W, fc1_b.reshape(1, H),
                fc2_W.reshape(1, H), fc2_b.reshape(1, 1))
    return out


# speedup vs baseline: 1.0392x; 1.0392x over previous
import jax
import jax.numpy as jnp
from jax import lax
from jax.experimental import pallas as pl
from jax.experimental.pallas import tpu as pltpu
from jax.experimental.pallas import tpu_sc as plsc

N = 10000
C = 128
H = 64
E = 320000
NSUB = 16
EPT = E // NSUB
CH = 80
NCHUNK = EPT // CH
BN = 2000
NB = N // BN


def _dot(a, b):
    return jnp.dot(a, b, preferred_element_type=jnp.float32)


def _dense_half(X, posp_ref, wpos_ref, bpos_ref, wsrc_ref, wlin_ref):
    P = _dot(posp_ref[...], wpos_ref[0])
    Pb = P + bpos_ref[0]
    S = _dot(X, wsrc_ref[0]) + P
    V = _dot(X, wlin_ref[0]) - P
    return Pb, S, V


def _phase1_out(i, posp_ref, wpos_ref, bpos_ref, sv_s, mn_s, t_ref, pb_ref):
    P = _dot(posp_ref[...], wpos_ref[0])
    pb_ref[0] = P + bpos_ref[0]
    SV = sv_s[pl.ds(i * BN, BN), :]
    Es = jnp.exp(mn_s[0:1, :H] - SV[:, :H])
    t_ref[...] = jnp.concatenate([Es, Es * SV[:, H:]], axis=1)


def _prep_body(x_ref, posp_ref, wpos_ref, bpos_ref, wsrc_ref, wlin_ref,
               t_ref, pb_ref, sv_s, mn_s):
    p = pl.program_id(1)
    i = pl.program_id(2)

    @pl.when(p == 0)
    def _():
        _, S, V = _dense_half(x_ref[...], posp_ref, wpos_ref, bpos_ref,
                              wsrc_ref, wlin_ref)
        sv_s[pl.ds(i * BN, BN), :] = jnp.concatenate([S, V], axis=1)
        cmn = jnp.min(S, axis=0, keepdims=True)

        @pl.when(i == 0)
        def _():
            mn_s[0:1, :H] = cmn

        @pl.when(i > 0)
        def _():
            mn_s[0:1, :H] = jnp.minimum(mn_s[0:1, :H], cmn)

    @pl.when(p == 1)
    def _():
        _phase1_out(i, posp_ref, wpos_ref, bpos_ref, sv_s, mn_s, t_ref,
                    pb_ref)


_PREP_SPECS = dict(
    grid=(2, 2, NB),
    out_specs=[
        pl.BlockSpec((BN, C), lambda h, p, i: (h * NB + i, 0)),
        pl.BlockSpec((1, BN, H), lambda h, p, i: (h, i, 0)),
    ],
    out_shape=[
        jax.ShapeDtypeStruct((2 * N, C), jnp.float32),
        jax.ShapeDtypeStruct((2, N, H), jnp.float32),
    ],
    scratch_shapes=[
        pltpu.VMEM((N, C), jnp.float32),
        pltpu.VMEM((8, C), jnp.float32),
    ],
    compiler_params=pltpu.CompilerParams(
        dimension_semantics=("arbitrary", "arbitrary", "arbitrary")),
)

_W_SPECS = [
    pl.BlockSpec((BN, 8), lambda h, p, i: (i, 0)),
    pl.BlockSpec((1, 8, H), lambda h, p, i: (h, 0, 0)),
    pl.BlockSpec((1, 1, H), lambda h, p, i: (h, 0, 0)),
    pl.BlockSpec((1, C, H), lambda h, p, i: (h, 0, 0)),
    pl.BlockSpec((1, C, H), lambda h, p, i: (h, 0, 0)),
]


def _prep(x, posp, wposp, bpos2, wsrc, wlin):
    return pl.pallas_call(
        _prep_body,
        in_specs=[pl.BlockSpec((BN, C), lambda h, p, i: (i, 0))] + _W_SPECS,
        **_PREP_SPECS,
    )(x, posp, wposp, bpos2, wsrc, wlin)


def _finalize_h(acca_ref, accb_ref, pb_ref):
    den = jnp.concatenate([acca_ref[:, :H], accb_ref[:, :H]], axis=1)
    num = jnp.concatenate([acca_ref[:, H:], accb_ref[:, H:]], axis=1)
    pbf = jnp.concatenate([pb_ref[0], pb_ref[1]], axis=1)
    hidden = jnp.where(den > 0.0, num / den + pbf, 0.0)
    return jnp.maximum(hidden, 0.0)


def _mid_body(acca_ref, accb_ref, pb1_ref, posp_ref, wpos_ref, bpos_ref,
              wsrc_ref, wlin_ref, t_ref, pb_ref, sv_s, mn_s):
    p = pl.program_id(1)
    i = pl.program_id(2)

    @pl.when(p == 0)
    def _():
        X = _finalize_h(acca_ref, accb_ref, pb1_ref)
        _, S, V = _dense_half(X, posp_ref, wpos_ref, bpos_ref, wsrc_ref,
                              wlin_ref)
        sv_s[pl.ds(i * BN, BN), :] = jnp.concatenate([S, V], axis=1)
        cmn = jnp.min(S, axis=0, keepdims=True)

        @pl.when(i == 0)
        def _():
            mn_s[0:1, :H] = cmn

        @pl.when(i > 0)
        def _():
            mn_s[0:1, :H] = jnp.minimum(mn_s[0:1, :H], cmn)

    @pl.when(p == 1)
    def _():
        _phase1_out(i, posp_ref, wpos_ref, bpos_ref, sv_s, mn_s, t_ref,
                    pb_ref)


def _mid(acc1, pb1, posp, wposp, bpos2, wsrc, wlin):
    return pl.pallas_call(
        _mid_body,
        in_specs=[
            pl.BlockSpec((BN, C), lambda h, p, i: (i, 0)),
            pl.BlockSpec((BN, C), lambda h, p, i: (NB + i, 0)),
            pl.BlockSpec((2, BN, H), lambda h, p, i: (0, i, 0)),
        ] + _W_SPECS,
        **_PREP_SPECS,
    )(acc1, acc1, pb1, posp, wposp, bpos2, wsrc, wlin)


def _head_body(acca_ref, accb_ref, pb2_ref, w1_ref, b1_ref, w2t_ref, b2_ref,
               out_ref):
    hidden = _finalize_h(acca_ref, accb_ref, pb2_ref)
    f = _dot(hidden, w1_ref[...])
    f = jnp.maximum(f + b1_ref[...], 0.0)
    out_ref[...] = (jnp.sum(f * w2t_ref[...], axis=1, keepdims=True)
                    + b2_ref[...])


def _head(acc2, pb2, fc1w, fc1b2, fc2wt, fc2b2):
    return pl.pallas_call(
        _head_body,
        grid=(NB,),
        in_specs=[
            pl.BlockSpec((BN, C), lambda i: (i, 0)),
            pl.BlockSpec((BN, C), lambda i: (NB + i, 0)),
            pl.BlockSpec((2, BN, H), lambda i: (0, i, 0)),
            pl.BlockSpec((C, H), lambda i: (0, 0)),
            pl.BlockSpec((1, H), lambda i: (0, 0)),
            pl.BlockSpec((1, H), lambda i: (0, 0)),
            pl.BlockSpec((1, 1), lambda i: (0, 0)),
        ],
        out_specs=pl.BlockSpec((BN, 1), lambda i: (i, 0)),
        out_shape=jax.ShapeDtypeStruct((N, 1), jnp.float32),
    )(acc2, acc2, pb2, fc1w, fc1b2, fc2wt, fc2b2)


def _edge_body(t_hbm, dst_hbm, src_hbm, z_hbm, acc_hbm, shared, dstv, idxv,
               rows0, rows1, sem0, sem1):
    c = lax.axis_index("c")
    s = lax.axis_index("s")
    coff = c * N
    rbase = s * 640

    @pl.when(s < 15)
    def _():
        pltpu.sync_copy(z_hbm.at[pl.ds(rbase, 640)],
                        shared.at[pl.ds(rbase, 640)])

    @pl.when(s == 15)
    def _():
        pltpu.sync_copy(z_hbm.at[pl.ds(9600, 400)],
                        shared.at[pl.ds(9600, 400)])

    plsc.subcore_barrier()

    def gather(j, rows, sem):
        return pltpu.async_copy(
            t_hbm.at[idxv.at[pl.ds(j * CH, CH)]], rows, sem)

    def wait(j, rows, sem):
        pltpu.make_async_copy(
            t_hbm.at[idxv.at[pl.ds(j * CH, CH)]], rows, sem).wait()

    def scatter(j, rows):
        pltpu.sync_copy(rows, shared.at[dstv.at[j]], add=True)

    NC2 = NCHUNK // 2
    NPAIR = NC2 // 2

    def stage_body(hh, carry):
        pltpu.sync_copy(dst_hbm.at[s, hh], dstv)
        pltpu.sync_copy(src_hbm.at[c, s, hh], idxv)

        gather(0, rows0, sem0)

        def pair_body(k, cc):
            j0 = 2 * k
            j1 = j0 + 1
            gather(j1, rows1, sem1)
            wait(j0, rows0, sem0)
            scatter(j0, rows0)

            @pl.when(k < NPAIR - 1)
            def _():
                gather(j0 + 2, rows0, sem0)

            wait(j1, rows1, sem1)
            scatter(j1, rows1)
            return cc

        lax.fori_loop(0, NPAIR, pair_body, 0)
        gather(NC2 - 1, rows0, sem0)
        wait(NC2 - 1, rows0, sem0)
        scatter(NC2 - 1, rows0)
        return carry

    lax.fori_loop(0, 2, stage_body, 0)
    plsc.subcore_barrier()

    @pl.when(s < 15)
    def _():
        pltpu.sync_copy(shared.at[pl.ds(rbase, 640)],
                        acc_hbm.at[pl.ds(coff + rbase, 640)])

    @pl.when(s == 15)
    def _():
        pltpu.sync_copy(shared.at[pl.ds(9600, 400)],
                        acc_hbm.at[pl.ds(coff + 9600, 400)])


def _edge(tstk, dst3, src2, zrows):
    f = pl.kernel(
        _edge_body,
        out_type=jax.ShapeDtypeStruct((2 * N, C), jnp.float32),
        mesh=plsc.VectorSubcoreMesh(core_axis_name="c", subcore_axis_name="s"),
        scratch_types=[
            pltpu.VMEM_SHARED((N, C), jnp.float32),
            pltpu.VMEM((NCHUNK // 2, CH), jnp.int32),
            pltpu.VMEM((EPT // 2,), jnp.int32),
            pltpu.VMEM((CH, C), jnp.float32),
            pltpu.VMEM((CH, C), jnp.float32),
            pltpu.SemaphoreType.DMA,
            pltpu.SemaphoreType.DMA,
        ],
    )
    return f(tstk, dst3, src2, zrows)


def kernel(x, pos, edge_index, batch, c1_Wpos, c1_bpos, c1_Wsrc, c1_Wdst,
           c1_Wlin, c2_Wpos, c2_bpos, c2_Wsrc, c2_Wdst, c2_Wlin, fc1_W,
           fc1_b, fc2_W, fc2_b):
    def _half(w):
        return jnp.stack([w[:, :H], w[:, H:]])

    src_r = edge_index[0].reshape(NSUB, 2, EPT // 2)
    src_a = jnp.stack([src_r, src_r + N])
    dst_a = edge_index[1].reshape(NSUB, 2, NCHUNK // 2, CH)
    posp = jnp.pad(pos, ((0, 0), (0, 5)))
    w1p = _half(jnp.pad(c1_Wpos, ((0, 5), (0, 0))))
    w2p = _half(jnp.pad(c2_Wpos, ((0, 5), (0, 0))))
    b1_2 = c1_bpos.reshape(2, 1, H)
    b2_2 = c2_bpos.reshape(2, 1, H)
    zrows = jnp.zeros((N, C), jnp.float32)

    t1, pb1 = _prep(x, posp, w1p, b1_2, _half(c1_Wsrc), _half(c1_Wlin))
    acc1 = _edge(t1, dst_a, src_a, zrows)
    t2, pb2 = _mid(acc1, pb1, posp, w2p, b2_2, _half(c2_Wsrc), _half(c2_Wlin))
    acc2 = _edge(t2, dst_a, src_a, zrows)
    out = _head(acc2, pb2, fc1_W, fc1_b.reshape(1, H),
                fc2_W.reshape(1, H), fc2_b.reshape(1, 1))
    return out


# speedup vs baseline: 1.0883x; 1.0472x over previous
import jax
import jax.numpy as jnp
from jax import lax
from jax.experimental import pallas as pl
from jax.experimental.pallas import tpu as pltpu
from jax.experimental.pallas import tpu_sc as plsc

N = 10000
C = 128
H = 64
E = 320000
NSUB = 16
EPT = E // NSUB
CH = 80
NCHUNK = EPT // CH
BN = 2000
NB = N // BN


def _dot(a, b):
    return jnp.dot(a, b, preferred_element_type=jnp.float32)


def _dense_half(X, posp_ref, wpos_ref, bpos_ref, wsrc_ref, wlin_ref):
    P = _dot(posp_ref[...], wpos_ref[0])
    Pb = P + bpos_ref[0]
    S = _dot(X, wsrc_ref[0]) + P
    V = _dot(X, wlin_ref[0]) - P
    return Pb, S, V


def _phase1_out(i, posp_ref, wpos_ref, bpos_ref, sv_s, mn_s, t_ref, pb_ref):
    P = _dot(posp_ref[...], wpos_ref[0])
    pb_ref[0] = P + bpos_ref[0]
    SV = sv_s[pl.ds(i * BN, BN), :]
    Es = jnp.exp(mn_s[0:1, :H] - SV[:, :H])
    t_ref[...] = jnp.concatenate([Es, Es * SV[:, H:]], axis=1)


def _prep_body(x_ref, posp_ref, wpos_ref, bpos_ref, wsrc_ref, wlin_ref,
               t_ref, pb_ref, sv_s, mn_s):
    p = pl.program_id(1)
    i = pl.program_id(2)

    @pl.when(p == 0)
    def _():
        _, S, V = _dense_half(x_ref[...], posp_ref, wpos_ref, bpos_ref,
                              wsrc_ref, wlin_ref)
        sv_s[pl.ds(i * BN, BN), :] = jnp.concatenate([S, V], axis=1)
        cmn = jnp.min(S, axis=0, keepdims=True)

        @pl.when(i == 0)
        def _():
            mn_s[0:1, :H] = cmn

        @pl.when(i > 0)
        def _():
            mn_s[0:1, :H] = jnp.minimum(mn_s[0:1, :H], cmn)

    @pl.when(p == 1)
    def _():
        _phase1_out(i, posp_ref, wpos_ref, bpos_ref, sv_s, mn_s, t_ref,
                    pb_ref)


_PREP_SPECS = dict(
    grid=(2, 2, NB),
    out_specs=[
        pl.BlockSpec((BN, C),
                     lambda h, p, i: (h * NB + jnp.where(p == 0, 0, i), 0)),
        pl.BlockSpec((1, BN, H),
                     lambda h, p, i: (h, jnp.where(p == 0, 0, i), 0)),
    ],
    out_shape=[
        jax.ShapeDtypeStruct((2 * N, C), jnp.float32),
        jax.ShapeDtypeStruct((2, N, H), jnp.float32),
    ],
    scratch_shapes=[
        pltpu.VMEM((N, C), jnp.float32),
        pltpu.VMEM((8, C), jnp.float32),
    ],
    compiler_params=pltpu.CompilerParams(
        dimension_semantics=("arbitrary", "arbitrary", "arbitrary")),
)

_W_SPECS = [
    pl.BlockSpec((BN, 8), lambda h, p, i: (i, 0)),
    pl.BlockSpec((1, 8, H), lambda h, p, i: (h, 0, 0)),
    pl.BlockSpec((1, 1, H), lambda h, p, i: (h, 0, 0)),
    pl.BlockSpec((1, C, H), lambda h, p, i: (h, 0, 0)),
    pl.BlockSpec((1, C, H), lambda h, p, i: (h, 0, 0)),
]


def _prep(x, posp, wposp, bpos2, wsrc, wlin):
    return pl.pallas_call(
        _prep_body,
        in_specs=[pl.BlockSpec(
            (BN, C), lambda h, p, i: (jnp.where(p == 0, i, NB - 1), 0))
        ] + _W_SPECS,
        **_PREP_SPECS,
    )(x, posp, wposp, bpos2, wsrc, wlin)


def _finalize_h(acca_ref, accb_ref, pb_ref):
    den = jnp.concatenate([acca_ref[:, :H], accb_ref[:, :H]], axis=1)
    num = jnp.concatenate([acca_ref[:, H:], accb_ref[:, H:]], axis=1)
    pbf = jnp.concatenate([pb_ref[0], pb_ref[1]], axis=1)
    hidden = jnp.where(den > 0.0, num / den + pbf, 0.0)
    return jnp.maximum(hidden, 0.0)


def _mid_body(acca_ref, accb_ref, pb1_ref, posp_ref, wpos_ref, bpos_ref,
              wsrc_ref, wlin_ref, t_ref, pb_ref, sv_s, mn_s):
    p = pl.program_id(1)
    i = pl.program_id(2)

    @pl.when(p == 0)
    def _():
        X = _finalize_h(acca_ref, accb_ref, pb1_ref)
        _, S, V = _dense_half(X, posp_ref, wpos_ref, bpos_ref, wsrc_ref,
                              wlin_ref)
        sv_s[pl.ds(i * BN, BN), :] = jnp.concatenate([S, V], axis=1)
        cmn = jnp.min(S, axis=0, keepdims=True)

        @pl.when(i == 0)
        def _():
            mn_s[0:1, :H] = cmn

        @pl.when(i > 0)
        def _():
            mn_s[0:1, :H] = jnp.minimum(mn_s[0:1, :H], cmn)

    @pl.when(p == 1)
    def _():
        _phase1_out(i, posp_ref, wpos_ref, bpos_ref, sv_s, mn_s, t_ref,
                    pb_ref)


def _mid(acc1, pb1, posp, wposp, bpos2, wsrc, wlin):
    return pl.pallas_call(
        _mid_body,
        in_specs=[
            pl.BlockSpec((BN, C),
                         lambda h, p, i: (jnp.where(p == 0, i, NB - 1), 0)),
            pl.BlockSpec((BN, C),
                         lambda h, p, i: (NB + jnp.where(p == 0, i, NB - 1),
                                          0)),
            pl.BlockSpec((2, BN, H),
                         lambda h, p, i: (0, jnp.where(p == 0, i, NB - 1),
                                          0)),
        ] + _W_SPECS,
        **_PREP_SPECS,
    )(acc1, acc1, pb1, posp, wposp, bpos2, wsrc, wlin)


def _head_body(acca_ref, accb_ref, pb2_ref, w1_ref, b1_ref, w2t_ref, b2_ref,
               out_ref):
    hidden = _finalize_h(acca_ref, accb_ref, pb2_ref)
    f = _dot(hidden, w1_ref[...])
    f = jnp.maximum(f + b1_ref[...], 0.0)
    out_ref[...] = (jnp.sum(f * w2t_ref[...], axis=1, keepdims=True)
                    + b2_ref[...])


def _head(acc2, pb2, fc1w, fc1b2, fc2wt, fc2b2):
    return pl.pallas_call(
        _head_body,
        grid=(NB,),
        in_specs=[
            pl.BlockSpec((BN, C), lambda i: (i, 0)),
            pl.BlockSpec((BN, C), lambda i: (NB + i, 0)),
            pl.BlockSpec((2, BN, H), lambda i: (0, i, 0)),
            pl.BlockSpec((C, H), lambda i: (0, 0)),
            pl.BlockSpec((1, H), lambda i: (0, 0)),
            pl.BlockSpec((1, H), lambda i: (0, 0)),
            pl.BlockSpec((1, 1), lambda i: (0, 0)),
        ],
        out_specs=pl.BlockSpec((BN, 1), lambda i: (i, 0)),
        out_shape=jax.ShapeDtypeStruct((N, 1), jnp.float32),
    )(acc2, acc2, pb2, fc1w, fc1b2, fc2wt, fc2b2)


def _edge_body(t_hbm, dst_hbm, src_hbm, acc_hbm, shared, dstv, idxv,
               rows0, rows1, sem0, sem1):
    c = lax.axis_index("c")
    s = lax.axis_index("s")
    coff = c * N
    rbase = s * 640

    def zrow(r, cc):
        for t in range(C // 16):
            rows0[r, pl.ds(t * 16, 16)] = jnp.zeros((16,), jnp.float32)
        return cc

    lax.fori_loop(0, CH, zrow, 0)

    @pl.when(s < 15)
    def _():
        def zcp(k, cc):
            pltpu.sync_copy(rows0, shared.at[pl.ds(rbase + k * CH, CH)])
            return cc

        lax.fori_loop(0, 640 // CH, zcp, 0)

    @pl.when(s == 15)
    def _():
        def zcp(k, cc):
            pltpu.sync_copy(rows0, shared.at[pl.ds(9600 + k * CH, CH)])
            return cc

        lax.fori_loop(0, 400 // CH, zcp, 0)

    plsc.subcore_barrier()

    def gather(j, rows, sem):
        return pltpu.async_copy(
            t_hbm.at[idxv.at[pl.ds(j * CH, CH)]], rows, sem)

    def wait(j, rows, sem):
        pltpu.make_async_copy(
            t_hbm.at[idxv.at[pl.ds(j * CH, CH)]], rows, sem).wait()

    def scatter(j, rows):
        pltpu.sync_copy(rows, shared.at[dstv.at[j]], add=True)

    NC2 = NCHUNK // 2
    NPAIR = NC2 // 2

    def stage_body(hh, carry):
        pltpu.sync_copy(dst_hbm.at[s, hh], dstv)
        pltpu.sync_copy(src_hbm.at[c, s, hh], idxv)

        gather(0, rows0, sem0)

        def pair_body(k, cc):
            j0 = 2 * k
            j1 = j0 + 1
            gather(j1, rows1, sem1)
            wait(j0, rows0, sem0)
            scatter(j0, rows0)

            @pl.when(k < NPAIR - 1)
            def _():
                gather(j0 + 2, rows0, sem0)

            wait(j1, rows1, sem1)
            scatter(j1, rows1)
            return cc

        lax.fori_loop(0, NPAIR, pair_body, 0)
        gather(NC2 - 1, rows0, sem0)
        wait(NC2 - 1, rows0, sem0)
        scatter(NC2 - 1, rows0)
        return carry

    lax.fori_loop(0, 2, stage_body, 0)
    plsc.subcore_barrier()

    @pl.when(s < 15)
    def _():
        pltpu.sync_copy(shared.at[pl.ds(rbase, 640)],
                        acc_hbm.at[pl.ds(coff + rbase, 640)])

    @pl.when(s == 15)
    def _():
        pltpu.sync_copy(shared.at[pl.ds(9600, 400)],
                        acc_hbm.at[pl.ds(coff + 9600, 400)])


def _edge(tstk, dst3, src2):
    f = pl.kernel(
        _edge_body,
        out_type=jax.ShapeDtypeStruct((2 * N, C), jnp.float32),
        mesh=plsc.VectorSubcoreMesh(core_axis_name="c", subcore_axis_name="s"),
        scratch_types=[
            pltpu.VMEM_SHARED((N, C), jnp.float32),
            pltpu.VMEM((NCHUNK // 2, CH), jnp.int32),
            pltpu.VMEM((EPT // 2,), jnp.int32),
            pltpu.VMEM((CH, C), jnp.float32),
            pltpu.VMEM((CH, C), jnp.float32),
            pltpu.SemaphoreType.DMA,
            pltpu.SemaphoreType.DMA,
        ],
    )
    return f(tstk, dst3, src2)


def kernel(x, pos, edge_index, batch, c1_Wpos, c1_bpos, c1_Wsrc, c1_Wdst,
           c1_Wlin, c2_Wpos, c2_bpos, c2_Wsrc, c2_Wdst, c2_Wlin, fc1_W,
           fc1_b, fc2_W, fc2_b):
    def _half(w):
        return jnp.stack([w[:, :H], w[:, H:]])

    src_r = edge_index[0].reshape(NSUB, 2, EPT // 2)
    src_a = jnp.stack([src_r, src_r + N])
    dst_a = edge_index[1].reshape(NSUB, 2, NCHUNK // 2, CH)
    posp = jnp.pad(pos, ((0, 0), (0, 5)))
    w1p = _half(jnp.pad(c1_Wpos, ((0, 5), (0, 0))))
    w2p = _half(jnp.pad(c2_Wpos, ((0, 5), (0, 0))))
    b1_2 = c1_bpos.reshape(2, 1, H)
    b2_2 = c2_bpos.reshape(2, 1, H)

    t1, pb1 = _prep(x, posp, w1p, b1_2, _half(c1_Wsrc), _half(c1_Wlin))
    acc1 = _edge(t1, dst_a, src_a)
    t2, pb2 = _mid(acc1, pb1, posp, w2p, b2_2, _half(c2_Wsrc), _half(c2_Wlin))
    acc2 = _edge(t2, dst_a, src_a)
    out = _head(acc2, pb2, fc1_W, fc1_b.reshape(1, H),
                fc2_W.reshape(1, H), fc2_b.reshape(1, 1))
    return out


# speedup vs baseline: 1.1281x; 1.0366x over previous
import jax
import jax.numpy as jnp
from jax import lax
from jax.experimental import pallas as pl
from jax.experimental.pallas import tpu as pltpu
from jax.experimental.pallas import tpu_sc as plsc

N = 10000
C = 128
H = 64
E = 320000
NSUB = 16
EPT = E // NSUB
CH = 80
NCHUNK = EPT // CH
BN = 5000
NB = N // BN


def _dot(a, b):
    return jnp.dot(a, b, preferred_element_type=jnp.float32)


def _dense_half(X, posp_ref, wpos_ref, bpos_ref, wsrc_ref, wlin_ref):
    P = _dot(posp_ref[...], wpos_ref[0])
    Pb = P + bpos_ref[0]
    S = _dot(X, wsrc_ref[0]) + P
    V = _dot(X, wlin_ref[0]) - P
    return Pb, S, V


def _phase1_out(i, posp_ref, wpos_ref, bpos_ref, sv_s, mn_s, t_ref, pb_ref):
    P = _dot(posp_ref[...], wpos_ref[0])
    pb_ref[0] = P + bpos_ref[0]
    SV = sv_s[pl.ds(i * BN, BN), :]
    Es = jnp.exp(mn_s[0:1, :H] - SV[:, :H])
    t_ref[...] = jnp.concatenate([Es, Es * SV[:, H:]], axis=1)


def _prep_body(x_ref, posp_ref, wpos_ref, bpos_ref, wsrc_ref, wlin_ref,
               t_ref, pb_ref, sv_s, mn_s):
    p = pl.program_id(1)
    i = pl.program_id(2)

    @pl.when(p == 0)
    def _():
        _, S, V = _dense_half(x_ref[...], posp_ref, wpos_ref, bpos_ref,
                              wsrc_ref, wlin_ref)
        sv_s[pl.ds(i * BN, BN), :] = jnp.concatenate([S, V], axis=1)
        cmn = jnp.min(S, axis=0, keepdims=True)

        @pl.when(i == 0)
        def _():
            mn_s[0:1, :H] = cmn

        @pl.when(i > 0)
        def _():
            mn_s[0:1, :H] = jnp.minimum(mn_s[0:1, :H], cmn)

    @pl.when(p == 1)
    def _():
        _phase1_out(i, posp_ref, wpos_ref, bpos_ref, sv_s, mn_s, t_ref,
                    pb_ref)


_PREP_SPECS = dict(
    grid=(2, 2, NB),
    out_specs=[
        pl.BlockSpec((BN, C),
                     lambda h, p, i: (h * NB + jnp.where(p == 0, 0, i), 0)),
        pl.BlockSpec((1, BN, H),
                     lambda h, p, i: (h, jnp.where(p == 0, 0, i), 0)),
    ],
    out_shape=[
        jax.ShapeDtypeStruct((2 * N, C), jnp.float32),
        jax.ShapeDtypeStruct((2, N, H), jnp.float32),
    ],
    scratch_shapes=[
        pltpu.VMEM((N, C), jnp.float32),
        pltpu.VMEM((8, C), jnp.float32),
    ],
    compiler_params=pltpu.CompilerParams(
        dimension_semantics=("arbitrary", "arbitrary", "arbitrary")),
)

_W_SPECS = [
    pl.BlockSpec((BN, 8), lambda h, p, i: (i, 0)),
    pl.BlockSpec((1, 8, H), lambda h, p, i: (h, 0, 0)),
    pl.BlockSpec((1, 1, H), lambda h, p, i: (h, 0, 0)),
    pl.BlockSpec((1, C, H), lambda h, p, i: (h, 0, 0)),
    pl.BlockSpec((1, C, H), lambda h, p, i: (h, 0, 0)),
]


def _prep(x, posp, wposp, bpos2, wsrc, wlin):
    return pl.pallas_call(
        _prep_body,
        in_specs=[pl.BlockSpec(
            (BN, C), lambda h, p, i: (jnp.where(p == 0, i, NB - 1), 0))
        ] + _W_SPECS,
        **_PREP_SPECS,
    )(x, posp, wposp, bpos2, wsrc, wlin)


def _finalize_h(acca_ref, accb_ref, pb_ref):
    den = jnp.concatenate([acca_ref[:, :H], accb_ref[:, :H]], axis=1)
    num = jnp.concatenate([acca_ref[:, H:], accb_ref[:, H:]], axis=1)
    pbf = jnp.concatenate([pb_ref[0], pb_ref[1]], axis=1)
    hidden = jnp.where(den > 0.0, num / den + pbf, 0.0)
    return jnp.maximum(hidden, 0.0)


def _mid_body(acca_ref, accb_ref, pb1_ref, posp_ref, wpos_ref, bpos_ref,
              wsrc_ref, wlin_ref, t_ref, pb_ref, sv_s, mn_s):
    p = pl.program_id(1)
    i = pl.program_id(2)

    @pl.when(p == 0)
    def _():
        X = _finalize_h(acca_ref, accb_ref, pb1_ref)
        _, S, V = _dense_half(X, posp_ref, wpos_ref, bpos_ref, wsrc_ref,
                              wlin_ref)
        sv_s[pl.ds(i * BN, BN), :] = jnp.concatenate([S, V], axis=1)
        cmn = jnp.min(S, axis=0, keepdims=True)

        @pl.when(i == 0)
        def _():
            mn_s[0:1, :H] = cmn

        @pl.when(i > 0)
        def _():
            mn_s[0:1, :H] = jnp.minimum(mn_s[0:1, :H], cmn)

    @pl.when(p == 1)
    def _():
        _phase1_out(i, posp_ref, wpos_ref, bpos_ref, sv_s, mn_s, t_ref,
                    pb_ref)


def _mid(acc1, pb1, posp, wposp, bpos2, wsrc, wlin):
    return pl.pallas_call(
        _mid_body,
        in_specs=[
            pl.BlockSpec((BN, C),
                         lambda h, p, i: (jnp.where(p == 0, i, NB - 1), 0)),
            pl.BlockSpec((BN, C),
                         lambda h, p, i: (NB + jnp.where(p == 0, i, NB - 1),
                                          0)),
            pl.BlockSpec((2, BN, H),
                         lambda h, p, i: (0, jnp.where(p == 0, i, NB - 1),
                                          0)),
        ] + _W_SPECS,
        **_PREP_SPECS,
    )(acc1, acc1, pb1, posp, wposp, bpos2, wsrc, wlin)


def _head_body(acca_ref, accb_ref, pb2_ref, w1_ref, b1_ref, w2t_ref, b2_ref,
               out_ref):
    hidden = _finalize_h(acca_ref, accb_ref, pb2_ref)
    f = _dot(hidden, w1_ref[...])
    f = jnp.maximum(f + b1_ref[...], 0.0)
    out_ref[...] = (jnp.sum(f * w2t_ref[...], axis=1, keepdims=True)
                    + b2_ref[...])


def _head(acc2, pb2, fc1w, fc1b2, fc2wt, fc2b2):
    return pl.pallas_call(
        _head_body,
        grid=(NB,),
        in_specs=[
            pl.BlockSpec((BN, C), lambda i: (i, 0)),
            pl.BlockSpec((BN, C), lambda i: (NB + i, 0)),
            pl.BlockSpec((2, BN, H), lambda i: (0, i, 0)),
            pl.BlockSpec((C, H), lambda i: (0, 0)),
            pl.BlockSpec((1, H), lambda i: (0, 0)),
            pl.BlockSpec((1, H), lambda i: (0, 0)),
            pl.BlockSpec((1, 1), lambda i: (0, 0)),
        ],
        out_specs=pl.BlockSpec((BN, 1), lambda i: (i, 0)),
        out_shape=jax.ShapeDtypeStruct((N, 1), jnp.float32),
    )(acc2, acc2, pb2, fc1w, fc1b2, fc2wt, fc2b2)


def _edge_body(t_hbm, ei_hbm, acc_hbm, shared, dstv, idxv, dsc,
               rows0, rows1, sem0, sem1):
    c = lax.axis_index("c")
    s = lax.axis_index("s")
    coff = c * N
    rbase = s * 640

    def zrow(r, cc):
        for t in range(C // 16):
            rows0[r, pl.ds(t * 16, 16)] = jnp.zeros((16,), jnp.float32)
        return cc

    lax.fori_loop(0, CH, zrow, 0)

    @pl.when(s < 15)
    def _():
        def zcp(k, cc):
            pltpu.sync_copy(rows0, shared.at[pl.ds(rbase + k * CH, CH)])
            return cc

        lax.fori_loop(0, 640 // CH, zcp, 0)

    @pl.when(s == 15)
    def _():
        def zcp(k, cc):
            pltpu.sync_copy(rows0, shared.at[pl.ds(9600 + k * CH, CH)])
            return cc

        lax.fori_loop(0, 400 // CH, zcp, 0)

    plsc.subcore_barrier()

    def gather(j, rows, sem):
        return pltpu.async_copy(
            t_hbm.at[idxv.at[pl.ds(j * CH, CH)]], rows, sem)

    def wait(j, rows, sem):
        pltpu.make_async_copy(
            t_hbm.at[idxv.at[pl.ds(j * CH, CH)]], rows, sem).wait()

    def repack(j):
        for t in range(CH // 16):
            dsc[pl.ds(t * 16, 16)] = dstv[pl.ds(j * CH + t * 16, 16)]

    def scatter(rows):
        pltpu.sync_copy(rows, shared.at[dsc], add=True)

    NC2 = NCHUNK // 2
    NPAIR = NC2 // 2

    def stage_body(hh, carry):
        ebase = s * EPT + hh * (EPT // 2)
        pltpu.sync_copy(ei_hbm.at[pl.ds(E + ebase, EPT // 2)], dstv)
        pltpu.sync_copy(ei_hbm.at[pl.ds(ebase, EPT // 2)], idxv)

        def addoff(k, cc):
            idxv[pl.ds(k * 16, 16)] = idxv[pl.ds(k * 16, 16)] + coff
            return cc

        lax.fori_loop(0, (EPT // 2) // 16, addoff, 0)

        gather(0, rows0, sem0)

        def pair_body(k, cc):
            j0 = 2 * k
            j1 = j0 + 1
            gather(j1, rows1, sem1)
            repack(j0)
            wait(j0, rows0, sem0)
            scatter(rows0)

            @pl.when(k < NPAIR - 1)
            def _():
                gather(j0 + 2, rows0, sem0)

            repack(j1)
            wait(j1, rows1, sem1)
            scatter(rows1)
            return cc

        lax.fori_loop(0, NPAIR, pair_body, 0)
        gather(NC2 - 1, rows0, sem0)
        repack(NC2 - 1)
        wait(NC2 - 1, rows0, sem0)
        scatter(rows0)
        return carry

    lax.fori_loop(0, 2, stage_body, 0)
    plsc.subcore_barrier()

    @pl.when(s < 15)
    def _():
        pltpu.sync_copy(shared.at[pl.ds(rbase, 640)],
                        acc_hbm.at[pl.ds(coff + rbase, 640)])

    @pl.when(s == 15)
    def _():
        pltpu.sync_copy(shared.at[pl.ds(9600, 400)],
                        acc_hbm.at[pl.ds(coff + 9600, 400)])


def _edge(tstk, ei):
    f = pl.kernel(
        _edge_body,
        out_type=jax.ShapeDtypeStruct((2 * N, C), jnp.float32),
        mesh=plsc.VectorSubcoreMesh(core_axis_name="c", subcore_axis_name="s"),
        scratch_types=[
            pltpu.VMEM_SHARED((N, C), jnp.float32),
            pltpu.VMEM((EPT // 2,), jnp.int32),
            pltpu.VMEM((EPT // 2,), jnp.int32),
            pltpu.VMEM((CH,), jnp.int32),
            pltpu.VMEM((CH, C), jnp.float32),
            pltpu.VMEM((CH, C), jnp.float32),
            pltpu.SemaphoreType.DMA,
            pltpu.SemaphoreType.DMA,
        ],
    )
    return f(tstk, ei)


def kernel(x, pos, edge_index, batch, c1_Wpos, c1_bpos, c1_Wsrc, c1_Wdst,
           c1_Wlin, c2_Wpos, c2_bpos, c2_Wsrc, c2_Wdst, c2_Wlin, fc1_W,
           fc1_b, fc2_W, fc2_b):
    def _half(w):
        return jnp.stack([w[:, :H], w[:, H:]])

    posp = jnp.pad(pos, ((0, 0), (0, 5)))
    w1p = _half(jnp.pad(c1_Wpos, ((0, 5), (0, 0))))
    w2p = _half(jnp.pad(c2_Wpos, ((0, 5), (0, 0))))
    b1_2 = c1_bpos.reshape(2, 1, H)
    b2_2 = c2_bpos.reshape(2, 1, H)

    t1, pb1 = _prep(x, posp, w1p, b1_2, _half(c1_Wsrc), _half(c1_Wlin))
    ei_flat = edge_index.reshape(2 * E)
    acc1 = _edge(t1, ei_flat)
    t2, pb2 = _mid(acc1, pb1, posp, w2p, b2_2, _half(c2_Wsrc), _half(c2_Wlin))
    acc2 = _edge(t2, ei_flat)
    out = _head(acc2, pb2, fc1_W, fc1_b.reshape(1, H),
                fc2_W.reshape(1, H), fc2_b.reshape(1, 1))
    return out


# speedup vs baseline: 1.1635x; 1.0314x over previous
import jax
import jax.numpy as jnp
from jax import lax
from jax.experimental import pallas as pl
from jax.experimental.pallas import tpu as pltpu
from jax.experimental.pallas import tpu_sc as plsc

N = 10000
C = 128
H = 64
E = 320000
NSUB = 16
EPT = E // NSUB
CH = 80
NCHUNK = EPT // CH


def _dot(a, b):
    return jnp.dot(a, b, preferred_element_type=jnp.float32)


def _prep_body(x_ref, posp_ref, wpos_ref, bpos_ref, wsrc_ref, wlin_ref,
               t_ref, pb_ref):
    X = x_ref[...]
    P = _dot(posp_ref[...], wpos_ref[0])
    S = _dot(X, wsrc_ref[0]) + P
    V = _dot(X, wlin_ref[0]) - P
    mn = jnp.min(S, axis=0, keepdims=True)
    Es = jnp.exp(mn - S)
    t_ref[...] = jnp.concatenate([Es, Es * V], axis=1)
    pb_ref[0] = P + bpos_ref[0]


_PREP_SPECS = dict(
    grid=(2,),
    out_specs=[
        pl.BlockSpec((N, C), lambda h: (h, 0)),
        pl.BlockSpec((1, N, H), lambda h: (h, 0, 0)),
    ],
    out_shape=[
        jax.ShapeDtypeStruct((2 * N, C), jnp.float32),
        jax.ShapeDtypeStruct((2, N, H), jnp.float32),
    ],
    compiler_params=pltpu.CompilerParams(
        dimension_semantics=("arbitrary",),
        vmem_limit_bytes=64 * 1024 * 1024),
)

_W_SPECS = [
    pl.BlockSpec((N, 8), lambda h: (0, 0)),
    pl.BlockSpec((1, 8, H), lambda h: (h, 0, 0)),
    pl.BlockSpec((1, 1, H), lambda h: (h, 0, 0)),
    pl.BlockSpec((1, C, H), lambda h: (h, 0, 0)),
    pl.BlockSpec((1, C, H), lambda h: (h, 0, 0)),
]


def _prep(x, posp, wposp, bpos2, wsrc, wlin):
    return pl.pallas_call(
        _prep_body,
        in_specs=[pl.BlockSpec((N, C), lambda h: (0, 0))] + _W_SPECS,
        **_PREP_SPECS,
    )(x, posp, wposp, bpos2, wsrc, wlin)


def _finalize_h(acca_ref, accb_ref, pb_ref):
    den = jnp.concatenate([acca_ref[:, :H], accb_ref[:, :H]], axis=1)
    num = jnp.concatenate([acca_ref[:, H:], accb_ref[:, H:]], axis=1)
    pbf = jnp.concatenate([pb_ref[0], pb_ref[1]], axis=1)
    hidden = jnp.where(den > 0.0, num / den + pbf, 0.0)
    return jnp.maximum(hidden, 0.0)


def _mid_body(acca_ref, accb_ref, pb1_ref, posp_ref, wpos_ref, bpos_ref,
              wsrc_ref, wlin_ref, t_ref, pb_ref):
    X = _finalize_h(acca_ref, accb_ref, pb1_ref)
    P = _dot(posp_ref[...], wpos_ref[0])
    S = _dot(X, wsrc_ref[0]) + P
    V = _dot(X, wlin_ref[0]) - P
    mn = jnp.min(S, axis=0, keepdims=True)
    Es = jnp.exp(mn - S)
    t_ref[...] = jnp.concatenate([Es, Es * V], axis=1)
    pb_ref[0] = P + bpos_ref[0]


def _mid(acc1, pb1, posp, wposp, bpos2, wsrc, wlin):
    return pl.pallas_call(
        _mid_body,
        in_specs=[
            pl.BlockSpec((N, C), lambda h: (0, 0)),
            pl.BlockSpec((N, C), lambda h: (1, 0)),
            pl.BlockSpec((2, N, H), lambda h: (0, 0, 0)),
        ] + _W_SPECS,
        **_PREP_SPECS,
    )(acc1, acc1, pb1, posp, wposp, bpos2, wsrc, wlin)


def _head_body(acca_ref, accb_ref, pb2_ref, w1_ref, b1_ref, w2t_ref, b2_ref,
               out_ref):
    hidden = _finalize_h(acca_ref, accb_ref, pb2_ref)
    f = _dot(hidden, w1_ref[...])
    f = jnp.maximum(f + b1_ref[...], 0.0)
    out_ref[...] = (jnp.sum(f * w2t_ref[...], axis=1, keepdims=True)
                    + b2_ref[...])


def _head(acc2, pb2, fc1w, fc1b2, fc2wt, fc2b2):
    return pl.pallas_call(
        _head_body,
        grid=(1,),
        in_specs=[
            pl.BlockSpec((N, C), lambda i: (0, 0)),
            pl.BlockSpec((N, C), lambda i: (1, 0)),
            pl.BlockSpec((2, N, H), lambda i: (0, 0, 0)),
            pl.BlockSpec((C, H), lambda i: (0, 0)),
            pl.BlockSpec((1, H), lambda i: (0, 0)),
            pl.BlockSpec((1, H), lambda i: (0, 0)),
            pl.BlockSpec((1, 1), lambda i: (0, 0)),
        ],
        out_specs=pl.BlockSpec((N, 1), lambda i: (0, 0)),
        out_shape=jax.ShapeDtypeStruct((N, 1), jnp.float32),
    )(acc2, acc2, pb2, fc1w, fc1b2, fc2wt, fc2b2)


def _edge_body(t_hbm, ei_hbm, acc_hbm, shared, dstv, idxv, dsc,
               rows0, rows1, sem0, sem1):
    c = lax.axis_index("c")
    s = lax.axis_index("s")
    coff = c * N
    rbase = s * 640

    def zrow(r, cc):
        for t in range(C // 16):
            rows0[r, pl.ds(t * 16, 16)] = jnp.zeros((16,), jnp.float32)
        return cc

    lax.fori_loop(0, CH, zrow, 0)

    @pl.when(s < 15)
    def _():
        def zcp(k, cc):
            pltpu.sync_copy(rows0, shared.at[pl.ds(rbase + k * CH, CH)])
            return cc

        lax.fori_loop(0, 640 // CH, zcp, 0)

    @pl.when(s == 15)
    def _():
        def zcp(k, cc):
            pltpu.sync_copy(rows0, shared.at[pl.ds(9600 + k * CH, CH)])
            return cc

        lax.fori_loop(0, 400 // CH, zcp, 0)

    plsc.subcore_barrier()

    def gather(j, rows, sem):
        return pltpu.async_copy(
            t_hbm.at[idxv.at[pl.ds(j * CH, CH)]], rows, sem)

    def wait(j, rows, sem):
        pltpu.make_async_copy(
            t_hbm.at[idxv.at[pl.ds(j * CH, CH)]], rows, sem).wait()

    def repack(j):
        for t in range(CH // 16):
            dsc[pl.ds(t * 16, 16)] = dstv[pl.ds(j * CH + t * 16, 16)]

    def scatter(rows):
        pltpu.sync_copy(rows, shared.at[dsc], add=True)

    NC2 = NCHUNK // 2
    NPAIR = NC2 // 2

    def stage_body(hh, carry):
        ebase = s * EPT + hh * (EPT // 2)
        pltpu.sync_copy(ei_hbm.at[pl.ds(E + ebase, EPT // 2)], dstv)
        pltpu.sync_copy(ei_hbm.at[pl.ds(ebase, EPT // 2)], idxv)

        def addoff(k, cc):
            idxv[pl.ds(k * 16, 16)] = idxv[pl.ds(k * 16, 16)] + coff
            return cc

        lax.fori_loop(0, (EPT // 2) // 16, addoff, 0)

        gather(0, rows0, sem0)

        def pair_body(k, cc):
            j0 = 2 * k
            j1 = j0 + 1
            gather(j1, rows1, sem1)
            repack(j0)
            wait(j0, rows0, sem0)
            scatter(rows0)

            @pl.when(k < NPAIR - 1)
            def _():
                gather(j0 + 2, rows0, sem0)

            repack(j1)
            wait(j1, rows1, sem1)
            scatter(rows1)
            return cc

        lax.fori_loop(0, NPAIR, pair_body, 0)
        gather(NC2 - 1, rows0, sem0)
        repack(NC2 - 1)
        wait(NC2 - 1, rows0, sem0)
        scatter(rows0)
        return carry

    lax.fori_loop(0, 2, stage_body, 0)
    plsc.subcore_barrier()

    @pl.when(s < 15)
    def _():
        pltpu.sync_copy(shared.at[pl.ds(rbase, 640)],
                        acc_hbm.at[pl.ds(coff + rbase, 640)])

    @pl.when(s == 15)
    def _():
        pltpu.sync_copy(shared.at[pl.ds(9600, 400)],
                        acc_hbm.at[pl.ds(coff + 9600, 400)])


def _edge(tstk, ei):
    f = pl.kernel(
        _edge_body,
        out_type=jax.ShapeDtypeStruct((2 * N, C), jnp.float32),
        mesh=plsc.VectorSubcoreMesh(core_axis_name="c", subcore_axis_name="s"),
        scratch_types=[
            pltpu.VMEM_SHARED((N, C), jnp.float32),
            pltpu.VMEM((EPT // 2,), jnp.int32),
            pltpu.VMEM((EPT // 2,), jnp.int32),
            pltpu.VMEM((CH,), jnp.int32),
            pltpu.VMEM((CH, C), jnp.float32),
            pltpu.VMEM((CH, C), jnp.float32),
            pltpu.SemaphoreType.DMA,
            pltpu.SemaphoreType.DMA,
        ],
    )
    return f(tstk, ei)


def kernel(x, pos, edge_index, batch, c1_Wpos, c1_bpos, c1_Wsrc, c1_Wdst,
           c1_Wlin, c2_Wpos, c2_bpos, c2_Wsrc, c2_Wdst, c2_Wlin, fc1_W,
           fc1_b, fc2_W, fc2_b):
    def _half(w):
        return jnp.stack([w[:, :H], w[:, H:]])

    posp = jnp.pad(pos, ((0, 0), (0, 5)))
    w1p = _half(jnp.pad(c1_Wpos, ((0, 5), (0, 0))))
    w2p = _half(jnp.pad(c2_Wpos, ((0, 5), (0, 0))))
    b1_2 = c1_bpos.reshape(2, 1, H)
    b2_2 = c2_bpos.reshape(2, 1, H)
    ei_flat = edge_index.reshape(2 * E)

    t1, pb1 = _prep(x, posp, w1p, b1_2, _half(c1_Wsrc), _half(c1_Wlin))
    acc1 = _edge(t1, ei_flat)
    t2, pb2 = _mid(acc1, pb1, posp, w2p, b2_2, _half(c2_Wsrc), _half(c2_Wlin))
    acc2 = _edge(t2, ei_flat)
    out = _head(acc2, pb2, fc1_W, fc1_b.reshape(1, H),
                fc2_W.reshape(1, H), fc2_b.reshape(1, 1))
    return out


# speedup vs baseline: 1.1651x; 1.0014x over previous
import jax
import jax.numpy as jnp
from jax import lax
from jax.experimental import pallas as pl
from jax.experimental.pallas import tpu as pltpu
from jax.experimental.pallas import tpu_sc as plsc

N = 10000
C = 128
H = 64
E = 320000
NSUB = 16
EPT = E // NSUB
CH = 80
NCHUNK = EPT // CH


def _dot(a, b):
    return jnp.dot(a, b, preferred_element_type=jnp.float32)


def _prep_body(x_ref, posp_ref, wpos_ref, bpos_ref, wsrc_ref, wlin_ref,
               t_ref, pb_ref):
    X = x_ref[...]
    P = _dot(posp_ref[...], wpos_ref[0])
    S = _dot(X, wsrc_ref[0]) + P
    V = _dot(X, wlin_ref[0]) - P
    mn = jnp.min(S, axis=0, keepdims=True)
    Es = jnp.exp(mn - S)
    t_ref[...] = jnp.concatenate([Es, Es * V], axis=1)
    pb_ref[0] = P + bpos_ref[0]


_PREP_SPECS = dict(
    grid=(2,),
    out_specs=[
        pl.BlockSpec((N, C), lambda h: (h, 0)),
        pl.BlockSpec((1, N, H), lambda h: (h, 0, 0)),
    ],
    out_shape=[
        jax.ShapeDtypeStruct((2 * N, C), jnp.float32),
        jax.ShapeDtypeStruct((2, N, H), jnp.float32),
    ],
    compiler_params=pltpu.CompilerParams(
        dimension_semantics=("arbitrary",),
        vmem_limit_bytes=64 * 1024 * 1024),
)

_W_SPECS = [
    pl.BlockSpec((N, 3), lambda h: (0, 0)),
    pl.BlockSpec((1, 3, H), lambda h: (h, 0, 0)),
    pl.BlockSpec((1, 1, H), lambda h: (h, 0, 0)),
    pl.BlockSpec((1, C, H), lambda h: (h, 0, 0)),
    pl.BlockSpec((1, C, H), lambda h: (h, 0, 0)),
]


def _prep(x, posp, wposp, bpos2, wsrc, wlin):
    return pl.pallas_call(
        _prep_body,
        in_specs=[pl.BlockSpec((N, C), lambda h: (0, 0))] + _W_SPECS,
        **_PREP_SPECS,
    )(x, posp, wposp, bpos2, wsrc, wlin)


def _finalize_h(acca_ref, accb_ref, pb_ref):
    den = jnp.concatenate([acca_ref[:, :H], accb_ref[:, :H]], axis=1)
    num = jnp.concatenate([acca_ref[:, H:], accb_ref[:, H:]], axis=1)
    pbf = jnp.concatenate([pb_ref[0], pb_ref[1]], axis=1)
    hidden = jnp.where(den > 0.0, num / den + pbf, 0.0)
    return jnp.maximum(hidden, 0.0)


def _mid_body(acca_ref, accb_ref, pb1_ref, posp_ref, wpos_ref, bpos_ref,
              wsrc_ref, wlin_ref, t_ref, pb_ref):
    X = _finalize_h(acca_ref, accb_ref, pb1_ref)
    P = _dot(posp_ref[...], wpos_ref[0])
    S = _dot(X, wsrc_ref[0]) + P
    V = _dot(X, wlin_ref[0]) - P
    mn = jnp.min(S, axis=0, keepdims=True)
    Es = jnp.exp(mn - S)
    t_ref[...] = jnp.concatenate([Es, Es * V], axis=1)
    pb_ref[0] = P + bpos_ref[0]


def _mid(acc1, pb1, posp, wposp, bpos2, wsrc, wlin):
    return pl.pallas_call(
        _mid_body,
        in_specs=[
            pl.BlockSpec((N, C), lambda h: (0, 0)),
            pl.BlockSpec((N, C), lambda h: (1, 0)),
            pl.BlockSpec((2, N, H), lambda h: (0, 0, 0)),
        ] + _W_SPECS,
        **_PREP_SPECS,
    )(acc1, acc1, pb1, posp, wposp, bpos2, wsrc, wlin)


def _head_body(acca_ref, accb_ref, pb2_ref, w1_ref, b1_ref, w2t_ref, b2_ref,
               out_ref):
    hidden = _finalize_h(acca_ref, accb_ref, pb2_ref)
    f = _dot(hidden, w1_ref[...])
    f = jnp.maximum(f + b1_ref[...], 0.0)
    out_ref[...] = (jnp.sum(f * w2t_ref[...], axis=1, keepdims=True)
                    + b2_ref[...])


def _head(acc2, pb2, fc1w, fc1b2, fc2wt, fc2b2):
    return pl.pallas_call(
        _head_body,
        grid=(1,),
        in_specs=[
            pl.BlockSpec((N, C), lambda i: (0, 0)),
            pl.BlockSpec((N, C), lambda i: (1, 0)),
            pl.BlockSpec((2, N, H), lambda i: (0, 0, 0)),
            pl.BlockSpec((C, H), lambda i: (0, 0)),
            pl.BlockSpec((1, H), lambda i: (0, 0)),
            pl.BlockSpec((1, H), lambda i: (0, 0)),
            pl.BlockSpec((1, 1), lambda i: (0, 0)),
        ],
        out_specs=pl.BlockSpec((N, 1), lambda i: (0, 0)),
        out_shape=jax.ShapeDtypeStruct((N, 1), jnp.float32),
    )(acc2, acc2, pb2, fc1w, fc1b2, fc2wt, fc2b2)


def _edge_body(t_hbm, ei_hbm, acc_hbm, shared, dstv, idxv, dsc,
               rows0, rows1, sem0, sem1):
    c = lax.axis_index("c")
    s = lax.axis_index("s")
    coff = c * N
    rbase = s * 640

    def zrow(r, cc):
        for t in range(C // 16):
            rows0[r, pl.ds(t * 16, 16)] = jnp.zeros((16,), jnp.float32)
        return cc

    lax.fori_loop(0, CH, zrow, 0)

    @pl.when(s < 15)
    def _():
        def zcp(k, cc):
            pltpu.sync_copy(rows0, shared.at[pl.ds(rbase + k * CH, CH)])
            return cc

        lax.fori_loop(0, 640 // CH, zcp, 0)

    @pl.when(s == 15)
    def _():
        def zcp(k, cc):
            pltpu.sync_copy(rows0, shared.at[pl.ds(9600 + k * CH, CH)])
            return cc

        lax.fori_loop(0, 400 // CH, zcp, 0)

    plsc.subcore_barrier()

    def gather(j, rows, sem):
        return pltpu.async_copy(
            t_hbm.at[idxv.at[pl.ds(j * CH, CH)]], rows, sem)

    def wait(j, rows, sem):
        pltpu.make_async_copy(
            t_hbm.at[idxv.at[pl.ds(j * CH, CH)]], rows, sem).wait()

    def repack(j):
        for t in range(CH // 16):
            dsc[pl.ds(t * 16, 16)] = dstv[pl.ds(j * CH + t * 16, 16)]

    def scatter(rows):
        pltpu.sync_copy(rows, shared.at[dsc], add=True)

    NC2 = NCHUNK // 2
    NPAIR = NC2 // 2

    def stage_body(hh, carry):
        ebase = s * EPT + hh * (EPT // 2)
        pltpu.sync_copy(ei_hbm.at[pl.ds(E + ebase, EPT // 2)], dstv)
        pltpu.sync_copy(ei_hbm.at[pl.ds(ebase, EPT // 2)], idxv)

        def addoff(k, cc):
            idxv[pl.ds(k * 16, 16)] = idxv[pl.ds(k * 16, 16)] + coff
            return cc

        lax.fori_loop(0, (EPT // 2) // 16, addoff, 0)

        gather(0, rows0, sem0)

        def pair_body(k, cc):
            j0 = 2 * k
            j1 = j0 + 1
            gather(j1, rows1, sem1)
            repack(j0)
            wait(j0, rows0, sem0)
            scatter(rows0)

            @pl.when(k < NPAIR - 1)
            def _():
                gather(j0 + 2, rows0, sem0)

            repack(j1)
            wait(j1, rows1, sem1)
            scatter(rows1)
            return cc

        lax.fori_loop(0, NPAIR, pair_body, 0)
        gather(NC2 - 1, rows0, sem0)
        repack(NC2 - 1)
        wait(NC2 - 1, rows0, sem0)
        scatter(rows0)
        return carry

    lax.fori_loop(0, 2, stage_body, 0)
    plsc.subcore_barrier()

    @pl.when(s < 15)
    def _():
        pltpu.sync_copy(shared.at[pl.ds(rbase, 640)],
                        acc_hbm.at[pl.ds(coff + rbase, 640)])

    @pl.when(s == 15)
    def _():
        pltpu.sync_copy(shared.at[pl.ds(9600, 400)],
                        acc_hbm.at[pl.ds(coff + 9600, 400)])


def _edge(tstk, ei):
    f = pl.kernel(
        _edge_body,
        out_type=jax.ShapeDtypeStruct((2 * N, C), jnp.float32),
        mesh=plsc.VectorSubcoreMesh(core_axis_name="c", subcore_axis_name="s"),
        scratch_types=[
            pltpu.VMEM_SHARED((N, C), jnp.float32),
            pltpu.VMEM((EPT // 2,), jnp.int32),
            pltpu.VMEM((EPT // 2,), jnp.int32),
            pltpu.VMEM((CH,), jnp.int32),
            pltpu.VMEM((CH, C), jnp.float32),
            pltpu.VMEM((CH, C), jnp.float32),
            pltpu.SemaphoreType.DMA,
            pltpu.SemaphoreType.DMA,
        ],
    )
    return f(tstk, ei)


def kernel(x, pos, edge_index, batch, c1_Wpos, c1_bpos, c1_Wsrc, c1_Wdst,
           c1_Wlin, c2_Wpos, c2_bpos, c2_Wsrc, c2_Wdst, c2_Wlin, fc1_W,
           fc1_b, fc2_W, fc2_b):
    def _half(w):
        return jnp.stack([w[:, :H], w[:, H:]])

    posp = pos
    w1p = _half(c1_Wpos)
    w2p = _half(c2_Wpos)
    b1_2 = c1_bpos.reshape(2, 1, H)
    b2_2 = c2_bpos.reshape(2, 1, H)
    ei_flat = edge_index.reshape(2 * E)

    t1, pb1 = _prep(x, posp, w1p, b1_2, _half(c1_Wsrc), _half(c1_Wlin))
    acc1 = _edge(t1, ei_flat)
    t2, pb2 = _mid(acc1, pb1, posp, w2p, b2_2, _half(c2_Wsrc), _half(c2_Wlin))
    acc2 = _edge(t2, ei_flat)
    out = _head(acc2, pb2, fc1_W, fc1_b.reshape(1, H),
                fc2_W.reshape(1, H), fc2_b.reshape(1, 1))
    return out
